# Initial kernel scaffold; baseline (speedup 1.0000x reference)
import jax
import jax.numpy as jnp
from jax.experimental import pallas as pl


def kernel(x, edge_index, batch, clinical, W1, b1, g1, be1, W2, b2, g2, be2, W3, b3, g3, be3, Wc, bc):
    raise NotImplementedError("write your pallas kernel here")


# speedup vs baseline: 12.8724x; 12.8724x over previous
import functools

import jax
import jax.numpy as jnp
from jax import lax
from jax.experimental import pallas as pl
from jax.experimental.pallas import tpu as pltpu
from jax.experimental.pallas import tpu_sc as plsc

_N = 10000
_NPAD = 10240
_D = 128
_E = 320000
_EPAD = 331776
_CHUNK = 128
_NCORES = 2
_NSUB = 16
_EPC = _EPAD // _NCORES
_EPT = _EPC // _NSUB
_NCHUNKS = _EPT // _CHUNK
_RPT = _NPAD // _NSUB
_RCH = _RPT // _CHUNK
_EPS = 1e-5
_G = 64
_NC = 16

def _deg_body(dstp, onesr, zrows, out, idx_d, ones_v, zb, acc):
    c = lax.axis_index("c")
    s = lax.axis_index("s")
    pltpu.sync_copy(onesr, ones_v)
    pltpu.sync_copy(zrows, zb)
    for t in range(_RCH):
        pltpu.sync_copy(zb, acc.at[pl.ds((s * _RCH + t) * _CHUNK, _CHUNK)])
    plsc.subcore_barrier()

    def step(i, carry):
        off = c * _EPC + s * _EPT + i * _CHUNK
        pltpu.sync_copy(dstp.at[pl.ds(off, _CHUNK)], idx_d)
        pltpu.sync_copy(ones_v, acc.at[idx_d], add=True)
        return carry

    lax.fori_loop(0, _NCHUNKS, step, 0)
    plsc.subcore_barrier()
    for t in range(_RCH):
        r0 = (s * _RCH + t) * _CHUNK
        pltpu.sync_copy(acc.at[pl.ds(r0, _CHUNK)], zb)
        pltpu.sync_copy(zb, out.at[c, pl.ds(r0, _CHUNK)])


def _agg_body(table, srcp, dstp, zrows, out, idx_s, idx_d, rows, sem, acc):
    c = lax.axis_index("c")
    s = lax.axis_index("s")
    pltpu.sync_copy(zrows, rows)
    for t in range(_RCH):
        pltpu.sync_copy(rows, acc.at[pl.ds((s * _RCH + t) * _CHUNK, _CHUNK)])
    plsc.subcore_barrier()

    def step(i, carry):
        off = c * _EPC + s * _EPT + i * _CHUNK
        pltpu.sync_copy(srcp.at[pl.ds(off, _CHUNK)], idx_s)
        pltpu.sync_copy(dstp.at[pl.ds(off, _CHUNK)], idx_d)
        pltpu.async_copy(table.at[idx_s], rows, sem).wait()
        pltpu.sync_copy(rows, acc.at[idx_d], add=True)
        return carry

    lax.fori_loop(0, _NCHUNKS, step, 0)
    plsc.subcore_barrier()
    for t in range(_RCH):
        r0 = (s * _RCH + t) * _CHUNK
        pltpu.sync_copy(acc.at[pl.ds(r0, _CHUNK)], rows)
        pltpu.sync_copy(rows, out.at[c, pl.ds(r0, _CHUNK)])


@functools.lru_cache(maxsize=None)
def _sc_kernels():
    mesh = plsc.VectorSubcoreMesh(core_axis_name="c", subcore_axis_name="s")
    deg = pl.kernel(
        _deg_body,
        out_type=jax.ShapeDtypeStruct((_NCORES, _NPAD, _D), jnp.float32),
        mesh=mesh,
        scratch_types=[
            pltpu.VMEM((_CHUNK,), jnp.int32),
            pltpu.VMEM((_CHUNK, _D), jnp.float32),
            pltpu.VMEM((_CHUNK, _D), jnp.float32),
            pltpu.VMEM_SHARED((_NPAD, _D), jnp.float32),
        ],
    )
    agg = pl.kernel(
        _agg_body,
        out_type=jax.ShapeDtypeStruct((_NCORES, _NPAD, _D), jnp.float32),
        mesh=mesh,
        scratch_types=[
            pltpu.VMEM((_CHUNK,), jnp.int32),
            pltpu.VMEM((_CHUNK,), jnp.int32),
            pltpu.VMEM((_CHUNK, _D), jnp.float32),
            pltpu.SemaphoreType.DMA,
            pltpu.VMEM_SHARED((_NPAD, _D), jnp.float32),
        ],
    )
    return deg, agg


def _k0_body(x_ref, w_ref, degp_ref, h_ref, dn_ref):
    deg = degp_ref[0, :_N, 0:1] + degp_ref[1, :_N, 0:1]
    dn = lax.rsqrt(jnp.maximum(deg, 1.0))
    h = jnp.dot(x_ref[...], w_ref[...], preferred_element_type=jnp.float32)
    h_ref[...] = h * dn
    dn_ref[...] = dn


_k0 = pl.pallas_call(
    _k0_body,
    out_shape=[
        jax.ShapeDtypeStruct((_N, _D), jnp.float32),
        jax.ShapeDtypeStruct((_N, 1), jnp.float32),
    ],
)


def _mid_body(aggp_ref, dn_ref, b_ref, g_ref, be_ref, w_ref, out_ref):
    a = aggp_ref[0, :_N, :] + aggp_ref[1, :_N, :]
    dn = dn_ref[...]
    r = jnp.maximum(a * dn + b_ref[...], 0.0)
    mu = jnp.mean(r, axis=0, keepdims=True)
    var = jnp.mean(r * r, axis=0, keepdims=True) - mu * mu
    y = (r - mu) * lax.rsqrt(var + _EPS) * g_ref[...] + be_ref[...]
    out_ref[...] = jnp.dot(y, w_ref[...], preferred_element_type=jnp.float32) * dn


_mid = pl.pallas_call(
    _mid_body,
    out_shape=jax.ShapeDtypeStruct((_N, _D), jnp.float32),
)


def _fin_body(aggp_ref, dn_ref, b_ref, g_ref, be_ref, batch_ref, clin_ref,
              wc1_ref, wc2_ref, bc_ref, out_ref):
    a = aggp_ref[0, :_N, :] + aggp_ref[1, :_N, :]
    r = jnp.maximum(a * dn_ref[...] + b_ref[...], 0.0)
    mu = jnp.mean(r, axis=0, keepdims=True)
    var = jnp.mean(r * r, axis=0, keepdims=True) - mu * mu
    y = (r - mu) * lax.rsqrt(var + _EPS) * g_ref[...] + be_ref[...]
    gids = lax.broadcasted_iota(jnp.int32, (_G, _N), 0)
    oh = jnp.where(gids == batch_ref[0:1, :_N], 1.0, 0.0)
    sums = jnp.dot(oh, y, preferred_element_type=jnp.float32)
    cnt = jnp.sum(oh, axis=1, keepdims=True)
    pooled = sums / jnp.maximum(cnt, 1.0)
    out_ref[...] = (jnp.dot(pooled, wc1_ref[...], preferred_element_type=jnp.float32)
                    + jnp.dot(clin_ref[...], wc2_ref[...], preferred_element_type=jnp.float32)
                    + bc_ref[...])


_fin = pl.pallas_call(
    _fin_body,
    out_shape=jax.ShapeDtypeStruct((_G, _D), jnp.float32),
)


def kernel(x, edge_index, batch, clinical, W1, b1, g1, be1, W2, b2, g2, be2,
           W3, b3, g3, be3, Wc, bc):
    loop = jnp.arange(_N, dtype=jnp.int32)
    npad_e = _EPAD - (_E + _N)
    padi = jnp.arange(npad_e, dtype=jnp.int32)
    srcp = jnp.concatenate([edge_index[0], loop, padi % _N])
    dstp = jnp.concatenate([edge_index[1], loop, _N + padi % (_NPAD - _N)])
    batchp = jnp.concatenate(
        [batch, jnp.full((_NPAD - _N,), _G, jnp.int32)]).reshape(1, _NPAD)
    zrows = jnp.zeros((_CHUNK, _D), jnp.float32)
    onesr = jnp.ones((_CHUNK, _D), jnp.float32)
    wc1 = jnp.pad(Wc[:_D], ((0, 0), (0, _D - Wc.shape[1])))
    wc2 = jnp.pad(Wc[_D:], ((0, 0), (0, _D - Wc.shape[1])))
    bcp = jnp.pad(bc, (0, _D - bc.shape[0]))

    _deg_kernel, _agg_kernel = _sc_kernels()
    degp = _deg_kernel(dstp, onesr, zrows)
    h1p, dn = _k0(x, W1, degp)
    a1 = _agg_kernel(h1p, srcp, dstp, zrows)
    h2p = _mid(a1, dn, b1, g1, be1, W2)
    a2 = _agg_kernel(h2p, srcp, dstp, zrows)
    h3p = _mid(a2, dn, b2, g2, be2, W3)
    a3 = _agg_kernel(h3p, srcp, dstp, zrows)
    o = _fin(a3, dn, b3, g3, be3, batchp, clinical, wc1, wc2, bcp)
    return o[:, :Wc.shape[1]]


# speedup vs baseline: 20.2767x; 1.5752x over previous
import functools

import jax
import jax.numpy as jnp
from jax import lax
from jax.experimental import pallas as pl
from jax.experimental.pallas import tpu as pltpu
from jax.experimental.pallas import tpu_sc as plsc

_N = 10000
_NPAD = 10240
_D = 128
_E = 320000
_EPAD = 360448
_CHUNK = 128
_NCORES = 2
_NSUB = 16
_EPC = _EPAD // _NCORES
_EPT = _EPC // _NSUB
_NCHUNKS = _EPT // _CHUNK
_RPT = _NPAD // _NSUB
_RCH = _RPT // _CHUNK
_EPS = 1e-5
_G = 64
_NC = 16

def _deg_body(dstp, onesr, zrows, out, idx_d, ones_v, zb, acc):
    c = lax.axis_index("c")
    s = lax.axis_index("s")
    cb = (c * _NSUB + s) * _NCHUNKS
    pltpu.sync_copy(dstp.at[pl.ds(cb, _NCHUNKS)], idx_d)
    pltpu.sync_copy(onesr, ones_v)
    pltpu.sync_copy(zrows, zb)
    for t in range(_RCH):
        pltpu.sync_copy(zb, acc.at[pl.ds((s * _RCH + t) * _CHUNK, _CHUNK)])
    plsc.subcore_barrier()

    def step(i, carry):
        pltpu.sync_copy(ones_v, acc.at[idx_d.at[i]], add=True)
        return carry

    lax.fori_loop(0, _NCHUNKS, step, 0)
    plsc.subcore_barrier()
    for t in range(_RCH):
        r0 = (s * _RCH + t) * _CHUNK
        pltpu.sync_copy(acc.at[pl.ds(r0, _CHUNK)], zb)
        pltpu.sync_copy(zb, out.at[c, pl.ds(r0, _CHUNK)])


_GRP = 22
_NGRP = _NCHUNKS // _GRP


def _agg_body(table, srcp, dstp, zrows, out,
              is0, is1, id0, id1, rows0, rows1,
              sg0, sg1, si0, si1, acc):
    c = lax.axis_index("c")
    s = lax.axis_index("s")
    ebase = c * _EPC + s * _EPT
    pltpu.sync_copy(zrows, rows0)
    for t in range(_RCH):
        pltpu.sync_copy(rows0, acc.at[pl.ds((s * _RCH + t) * _CHUNK, _CHUNK)])
    plsc.subcore_barrier()

    isb = (is0, is1)
    idb = (id0, id1)
    rb = (rows0, rows1)
    sg = (sg0, sg1)
    si = (si0, si1)

    def group(q, carry):
        gbase = ebase + q * _GRP * _CHUNK
        pltpu.sync_copy(srcp.at[pl.ds(gbase, _CHUNK)], is0)
        pltpu.sync_copy(dstp.at[pl.ds(gbase, _CHUNK)], id0)
        gd = [pltpu.async_copy(table.at[is0], rows0, sg0)]
        pref = {1: (pltpu.async_copy(srcp.at[pl.ds(gbase + _CHUNK, _CHUNK)], is1, si1),
                    pltpu.async_copy(dstp.at[pl.ds(gbase + _CHUNK, _CHUNK)], id1, si1))}
        for k in range(_GRP):
            p = k & 1
            gd[k].wait()
            if k + 1 < _GRP:
                a, b = pref.pop(k + 1)
                a.wait()
                b.wait()
                gd.append(pltpu.async_copy(table.at[isb[1 - p]], rb[1 - p], sg[1 - p]))
            pltpu.sync_copy(rb[p], acc.at[idb[p]], add=True)
            if k + 2 < _GRP:
                off = gbase + (k + 2) * _CHUNK
                pref[k + 2] = (
                    pltpu.async_copy(srcp.at[pl.ds(off, _CHUNK)], isb[p], si[p]),
                    pltpu.async_copy(dstp.at[pl.ds(off, _CHUNK)], idb[p], si[p]))
        return carry

    lax.fori_loop(0, _NGRP, group, 0)
    plsc.subcore_barrier()
    for t in range(_RCH):
        r0 = (s * _RCH + t) * _CHUNK
        pltpu.sync_copy(acc.at[pl.ds(r0, _CHUNK)], rows0)
        pltpu.sync_copy(rows0, out.at[c, pl.ds(r0, _CHUNK)])


@functools.lru_cache(maxsize=None)
def _sc_kernels():
    mesh = plsc.VectorSubcoreMesh(core_axis_name="c", subcore_axis_name="s")
    deg = pl.kernel(
        _deg_body,
        out_type=jax.ShapeDtypeStruct((_NCORES, _NPAD, _D), jnp.float32),
        mesh=mesh,
        scratch_types=[
            pltpu.VMEM((_NCHUNKS, _CHUNK), jnp.int32),
            pltpu.VMEM((_CHUNK, _D), jnp.float32),
            pltpu.VMEM((_CHUNK, _D), jnp.float32),
            pltpu.VMEM_SHARED((_NPAD, _D), jnp.float32),
        ],
    )
    agg = pl.kernel(
        _agg_body,
        out_type=jax.ShapeDtypeStruct((_NCORES, _NPAD, _D), jnp.float32),
        mesh=mesh,
        scratch_types=[
            pltpu.VMEM((_CHUNK,), jnp.int32),
            pltpu.VMEM((_CHUNK,), jnp.int32),
            pltpu.VMEM((_CHUNK,), jnp.int32),
            pltpu.VMEM((_CHUNK,), jnp.int32),
            pltpu.VMEM((_CHUNK, _D), jnp.float32),
            pltpu.VMEM((_CHUNK, _D), jnp.float32),
            pltpu.SemaphoreType.DMA,
            pltpu.SemaphoreType.DMA,
            pltpu.SemaphoreType.DMA,
            pltpu.SemaphoreType.DMA,
            pltpu.VMEM_SHARED((_NPAD, _D), jnp.float32),
        ],
    )
    return deg, agg


def _k0_body(x_ref, w_ref, degp_ref, h_ref, dn_ref):
    deg = degp_ref[0, :_N, 0:1] + degp_ref[1, :_N, 0:1]
    dn = lax.rsqrt(jnp.maximum(deg, 1.0))
    h = jnp.dot(x_ref[...], w_ref[...], preferred_element_type=jnp.float32)
    h_ref[...] = h * dn
    dn_ref[...] = dn


_k0 = pl.pallas_call(
    _k0_body,
    out_shape=[
        jax.ShapeDtypeStruct((_N, _D), jnp.float32),
        jax.ShapeDtypeStruct((_N, 1), jnp.float32),
    ],
)


def _mid_body(aggp_ref, dn_ref, b_ref, g_ref, be_ref, w_ref, out_ref):
    a = aggp_ref[0, :_N, :] + aggp_ref[1, :_N, :]
    dn = dn_ref[...]
    r = jnp.maximum(a * dn + b_ref[...], 0.0)
    mu = jnp.mean(r, axis=0, keepdims=True)
    var = jnp.mean(r * r, axis=0, keepdims=True) - mu * mu
    y = (r - mu) * lax.rsqrt(var + _EPS) * g_ref[...] + be_ref[...]
    out_ref[...] = jnp.dot(y, w_ref[...], preferred_element_type=jnp.float32) * dn


_mid = pl.pallas_call(
    _mid_body,
    out_shape=jax.ShapeDtypeStruct((_N, _D), jnp.float32),
)


def _fin_body(aggp_ref, dn_ref, b_ref, g_ref, be_ref, batch_ref, clin_ref,
              wc1_ref, wc2_ref, bc_ref, out_ref):
    a = aggp_ref[0, :_N, :] + aggp_ref[1, :_N, :]
    r = jnp.maximum(a * dn_ref[...] + b_ref[...], 0.0)
    mu = jnp.mean(r, axis=0, keepdims=True)
    var = jnp.mean(r * r, axis=0, keepdims=True) - mu * mu
    y = (r - mu) * lax.rsqrt(var + _EPS) * g_ref[...] + be_ref[...]
    gids = lax.broadcasted_iota(jnp.int32, (_G, _N), 0)
    oh = jnp.where(gids == batch_ref[0:1, :_N], 1.0, 0.0)
    sums = jnp.dot(oh, y, preferred_element_type=jnp.float32)
    cnt = jnp.sum(oh, axis=1, keepdims=True)
    pooled = sums / jnp.maximum(cnt, 1.0)
    out_ref[...] = (jnp.dot(pooled, wc1_ref[...], preferred_element_type=jnp.float32)
                    + jnp.dot(clin_ref[...], wc2_ref[...], preferred_element_type=jnp.float32)
                    + bc_ref[...])


_fin = pl.pallas_call(
    _fin_body,
    out_shape=jax.ShapeDtypeStruct((_G, _D), jnp.float32),
)


def kernel(x, edge_index, batch, clinical, W1, b1, g1, be1, W2, b2, g2, be2,
           W3, b3, g3, be3, Wc, bc):
    loop = jnp.arange(_N, dtype=jnp.int32)
    npad_e = _EPAD - (_E + _N)
    padi = jnp.arange(npad_e, dtype=jnp.int32)
    srcp = jnp.concatenate([edge_index[0], loop, padi % _N])
    dstp = jnp.concatenate([edge_index[1], loop, _N + padi % (_NPAD - _N)])
    dstp2 = dstp.reshape(-1, _CHUNK)
    batchp = jnp.concatenate(
        [batch, jnp.full((_NPAD - _N,), _G, jnp.int32)]).reshape(1, _NPAD)
    zrows = jnp.zeros((_CHUNK, _D), jnp.float32)
    onesr = jnp.ones((_CHUNK, _D), jnp.float32)
    wc1 = jnp.pad(Wc[:_D], ((0, 0), (0, _D - Wc.shape[1])))
    wc2 = jnp.pad(Wc[_D:], ((0, 0), (0, _D - Wc.shape[1])))
    bcp = jnp.pad(bc, (0, _D - bc.shape[0]))

    _deg_kernel, _agg_kernel = _sc_kernels()
    degp = _deg_kernel(dstp2, onesr, zrows)
    h1p, dn = _k0(x, W1, degp)
    a1 = _agg_kernel(h1p, srcp, dstp, zrows)
    h2p = _mid(a1, dn, b1, g1, be1, W2)
    a2 = _agg_kernel(h2p, srcp, dstp, zrows)
    h3p = _mid(a2, dn, b2, g2, be2, W3)
    a3 = _agg_kernel(h3p, srcp, dstp, zrows)
    o = _fin(a3, dn, b3, g3, be3, batchp, clinical, wc1, wc2, bcp)
    return o[:, :Wc.shape[1]]


# speedup vs baseline: 20.4900x; 1.0105x over previous
import functools

import jax
import jax.numpy as jnp
from jax import lax
from jax.experimental import pallas as pl
from jax.experimental.pallas import tpu as pltpu
from jax.experimental.pallas import tpu_sc as plsc

_N = 10000
_NPAD = 10240
_D = 128
_E = 320000
_EPAD = 360448
_CHUNK = 128
_NCORES = 2
_NSUB = 16
_EPC = _EPAD // _NCORES
_EPT = _EPC // _NSUB
_NCHUNKS = _EPT // _CHUNK
_RPT = _NPAD // _NSUB
_RCH = _RPT // _CHUNK
_EPS = 1e-5
_G = 64
_NC = 16

def _deg_body(dstp, onesr, zrows, out, idx_d, ones_v, zb, sem, acc):
    c = lax.axis_index("c")
    s = lax.axis_index("s")
    cb = (c * _NSUB + s) * _NCHUNKS
    pltpu.sync_copy(dstp.at[pl.ds(cb, _NCHUNKS)], idx_d)
    pltpu.sync_copy(onesr, ones_v)
    pltpu.sync_copy(zrows, zb)
    zd = [pltpu.async_copy(zb, acc.at[pl.ds((s * _RCH + t) * _CHUNK, _CHUNK)],
                           sem)
          for t in range(_RCH)]
    for d in zd:
        d.wait()
    plsc.subcore_barrier()

    def step(q, carry):
        ds = [pltpu.async_copy(ones_v, acc.at[idx_d.at[q * 8 + j]], sem,
                               add=True)
              for j in range(8)]
        for d in ds:
            d.wait()
        return carry

    lax.fori_loop(0, _NCHUNKS // 8, step, 0)
    plsc.subcore_barrier()
    rbuf = (zb, ones_v)
    rd = pltpu.async_copy(acc.at[pl.ds(s * _RCH * _CHUNK, _CHUNK)], zb, sem)
    for t in range(_RCH):
        p = t & 1
        rd.wait()
        if t + 1 < _RCH:
            rd = pltpu.async_copy(
                acc.at[pl.ds((s * _RCH + t + 1) * _CHUNK, _CHUNK)],
                rbuf[1 - p], sem)
        pltpu.sync_copy(rbuf[p], out.at[c, pl.ds((s * _RCH + t) * _CHUNK, _CHUNK)])


_GRP = 22
_NGRP = _NCHUNKS // _GRP


def _agg_body(table, srcp, dstp, zrows, out,
              is0, is1, id0, id1, rows0, rows1,
              sg0, sg1, si0, si1, acc):
    c = lax.axis_index("c")
    s = lax.axis_index("s")
    ebase = c * _EPC + s * _EPT
    pltpu.sync_copy(zrows, rows0)
    zd = [pltpu.async_copy(rows0, acc.at[pl.ds((s * _RCH + t) * _CHUNK, _CHUNK)],
                           sg0)
          for t in range(_RCH)]
    for d in zd:
        d.wait()
    plsc.subcore_barrier()

    isb = (is0, is1)
    idb = (id0, id1)
    rb = (rows0, rows1)
    sg = (sg0, sg1)
    si = (si0, si1)

    def group(q, carry):
        gbase = ebase + q * _GRP * _CHUNK
        pltpu.sync_copy(srcp.at[pl.ds(gbase, _CHUNK)], is0)
        pltpu.sync_copy(dstp.at[pl.ds(gbase, _CHUNK)], id0)
        gd = [pltpu.async_copy(table.at[is0], rows0, sg0)]
        pref = {1: (pltpu.async_copy(srcp.at[pl.ds(gbase + _CHUNK, _CHUNK)], is1, si1),
                    pltpu.async_copy(dstp.at[pl.ds(gbase + _CHUNK, _CHUNK)], id1, si1))}
        for k in range(_GRP):
            p = k & 1
            gd[k].wait()
            if k + 1 < _GRP:
                a, b = pref.pop(k + 1)
                a.wait()
                b.wait()
                gd.append(pltpu.async_copy(table.at[isb[1 - p]], rb[1 - p], sg[1 - p]))
            pltpu.sync_copy(rb[p], acc.at[idb[p]], add=True)
            if k + 2 < _GRP:
                off = gbase + (k + 2) * _CHUNK
                pref[k + 2] = (
                    pltpu.async_copy(srcp.at[pl.ds(off, _CHUNK)], isb[p], si[p]),
                    pltpu.async_copy(dstp.at[pl.ds(off, _CHUNK)], idb[p], si[p]))
        return carry

    lax.fori_loop(0, _NGRP, group, 0)
    plsc.subcore_barrier()
    rd = pltpu.async_copy(acc.at[pl.ds(s * _RCH * _CHUNK, _CHUNK)], rows0, sg0)
    for t in range(_RCH):
        p = t & 1
        rd.wait()
        if t + 1 < _RCH:
            rd = pltpu.async_copy(
                acc.at[pl.ds((s * _RCH + t + 1) * _CHUNK, _CHUNK)],
                rb[1 - p], sg[1 - p])
        pltpu.sync_copy(rb[p], out.at[c, pl.ds((s * _RCH + t) * _CHUNK, _CHUNK)])


@functools.lru_cache(maxsize=None)
def _sc_kernels():
    mesh = plsc.VectorSubcoreMesh(core_axis_name="c", subcore_axis_name="s")
    deg = pl.kernel(
        _deg_body,
        out_type=jax.ShapeDtypeStruct((_NCORES, _NPAD, _D), jnp.float32),
        mesh=mesh,
        scratch_types=[
            pltpu.VMEM((_NCHUNKS, _CHUNK), jnp.int32),
            pltpu.VMEM((_CHUNK, _D), jnp.float32),
            pltpu.VMEM((_CHUNK, _D), jnp.float32),
            pltpu.SemaphoreType.DMA,
            pltpu.VMEM_SHARED((_NPAD, _D), jnp.float32),
        ],
    )
    agg = pl.kernel(
        _agg_body,
        out_type=jax.ShapeDtypeStruct((_NCORES, _NPAD, _D), jnp.float32),
        mesh=mesh,
        scratch_types=[
            pltpu.VMEM((_CHUNK,), jnp.int32),
            pltpu.VMEM((_CHUNK,), jnp.int32),
            pltpu.VMEM((_CHUNK,), jnp.int32),
            pltpu.VMEM((_CHUNK,), jnp.int32),
            pltpu.VMEM((_CHUNK, _D), jnp.float32),
            pltpu.VMEM((_CHUNK, _D), jnp.float32),
            pltpu.SemaphoreType.DMA,
            pltpu.SemaphoreType.DMA,
            pltpu.SemaphoreType.DMA,
            pltpu.SemaphoreType.DMA,
            pltpu.VMEM_SHARED((_NPAD, _D), jnp.float32),
        ],
    )
    return deg, agg


def _k0_body(x_ref, w_ref, degp_ref, h_ref, dn_ref):
    deg = degp_ref[0, :_N, 0:1] + degp_ref[1, :_N, 0:1]
    dn = lax.rsqrt(jnp.maximum(deg, 1.0))
    h = jnp.dot(x_ref[...], w_ref[...], preferred_element_type=jnp.float32)
    h_ref[...] = h * dn
    dn_ref[...] = dn


_k0 = pl.pallas_call(
    _k0_body,
    out_shape=[
        jax.ShapeDtypeStruct((_N, _D), jnp.float32),
        jax.ShapeDtypeStruct((_N, 1), jnp.float32),
    ],
)


def _mid_body(aggp_ref, dn_ref, b_ref, g_ref, be_ref, w_ref, out_ref):
    a = aggp_ref[0, :_N, :] + aggp_ref[1, :_N, :]
    dn = dn_ref[...]
    r = jnp.maximum(a * dn + b_ref[...], 0.0)
    mu = jnp.mean(r, axis=0, keepdims=True)
    var = jnp.mean(r * r, axis=0, keepdims=True) - mu * mu
    y = (r - mu) * lax.rsqrt(var + _EPS) * g_ref[...] + be_ref[...]
    out_ref[...] = jnp.dot(y, w_ref[...], preferred_element_type=jnp.float32) * dn


_mid = pl.pallas_call(
    _mid_body,
    out_shape=jax.ShapeDtypeStruct((_N, _D), jnp.float32),
)


def _fin_body(aggp_ref, dn_ref, b_ref, g_ref, be_ref, batch_ref, clin_ref,
              wc1_ref, wc2_ref, bc_ref, out_ref):
    a = aggp_ref[0, :_N, :] + aggp_ref[1, :_N, :]
    r = jnp.maximum(a * dn_ref[...] + b_ref[...], 0.0)
    mu = jnp.mean(r, axis=0, keepdims=True)
    var = jnp.mean(r * r, axis=0, keepdims=True) - mu * mu
    y = (r - mu) * lax.rsqrt(var + _EPS) * g_ref[...] + be_ref[...]
    gids = lax.broadcasted_iota(jnp.int32, (_G, _N), 0)
    oh = jnp.where(gids == batch_ref[0:1, :_N], 1.0, 0.0)
    sums = jnp.dot(oh, y, preferred_element_type=jnp.float32)
    cnt = jnp.sum(oh, axis=1, keepdims=True)
    pooled = sums / jnp.maximum(cnt, 1.0)
    out_ref[...] = (jnp.dot(pooled, wc1_ref[...], preferred_element_type=jnp.float32)
                    + jnp.dot(clin_ref[...], wc2_ref[...], preferred_element_type=jnp.float32)
                    + bc_ref[...])


_fin = pl.pallas_call(
    _fin_body,
    out_shape=jax.ShapeDtypeStruct((_G, _D), jnp.float32),
)


def kernel(x, edge_index, batch, clinical, W1, b1, g1, be1, W2, b2, g2, be2,
           W3, b3, g3, be3, Wc, bc):
    loop = jnp.arange(_N, dtype=jnp.int32)
    npad_e = _EPAD - (_E + _N)
    padi = jnp.arange(npad_e, dtype=jnp.int32)
    srcp = jnp.concatenate([edge_index[0], loop, padi % _N])
    dstp = jnp.concatenate([edge_index[1], loop, _N + padi % (_NPAD - _N)])
    dstp2 = dstp.reshape(-1, _CHUNK)
    batchp = jnp.concatenate(
        [batch, jnp.full((_NPAD - _N,), _G, jnp.int32)]).reshape(1, _NPAD)
    zrows = jnp.zeros((_CHUNK, _D), jnp.float32)
    onesr = jnp.ones((_CHUNK, _D), jnp.float32)
    wc1 = jnp.pad(Wc[:_D], ((0, 0), (0, _D - Wc.shape[1])))
    wc2 = jnp.pad(Wc[_D:], ((0, 0), (0, _D - Wc.shape[1])))
    bcp = jnp.pad(bc, (0, _D - bc.shape[0]))

    _deg_kernel, _agg_kernel = _sc_kernels()
    degp = _deg_kernel(dstp2, onesr, zrows)
    h1p, dn = _k0(x, W1, degp)
    a1 = _agg_kernel(h1p, srcp, dstp, zrows)
    h2p = _mid(a1, dn, b1, g1, be1, W2)
    a2 = _agg_kernel(h2p, srcp, dstp, zrows)
    h3p = _mid(a2, dn, b2, g2, be2, W3)
    a3 = _agg_kernel(h3p, srcp, dstp, zrows)
    o = _fin(a3, dn, b3, g3, be3, batchp, clinical, wc1, wc2, bcp)
    return o[:, :Wc.shape[1]]


# speedup vs baseline: 20.5074x; 1.0009x over previous
import functools

import jax
import jax.numpy as jnp
from jax import lax
from jax.experimental import pallas as pl
from jax.experimental.pallas import tpu as pltpu
from jax.experimental.pallas import tpu_sc as plsc

_N = 10000
_NPAD = 10240
_D = 128
_E = 320000
_EPAD = 360448
_CHUNK = 128
_NCORES = 2
_NSUB = 16
_EPC = _EPAD // _NCORES
_EPT = _EPC // _NSUB
_NCHUNKS = _EPT // _CHUNK
_RPT = _NPAD // _NSUB
_RCH = _RPT // _CHUNK
_EPS = 1e-5
_G = 64
_NC = 16

def _deg_body(dstp, onesr, zrows, out, idx_d, ones_v, zb, sem, acc):
    c = lax.axis_index("c")
    s = lax.axis_index("s")
    cb = (c * _NSUB + s) * _NCHUNKS
    pltpu.sync_copy(dstp.at[pl.ds(cb, _NCHUNKS)], idx_d)
    pltpu.sync_copy(onesr, ones_v)
    pltpu.sync_copy(zrows, zb)
    zd = [pltpu.async_copy(zb, acc.at[pl.ds((s * _RCH + t) * _CHUNK, _CHUNK)],
                           sem)
          for t in range(_RCH)]
    for d in zd:
        d.wait()
    plsc.subcore_barrier()

    def step(q, carry):
        ds = [pltpu.async_copy(ones_v, acc.at[idx_d.at[q * 8 + j]], sem,
                               add=True)
              for j in range(8)]
        for d in ds:
            d.wait()
        return carry

    lax.fori_loop(0, _NCHUNKS // 8, step, 0)
    plsc.subcore_barrier()
    rbuf = (zb, ones_v)
    rd = pltpu.async_copy(acc.at[pl.ds(s * _RCH * _CHUNK, _CHUNK)], zb, sem)
    for t in range(_RCH):
        p = t & 1
        rd.wait()
        if t + 1 < _RCH:
            rd = pltpu.async_copy(
                acc.at[pl.ds((s * _RCH + t + 1) * _CHUNK, _CHUNK)],
                rbuf[1 - p], sem)
        pltpu.sync_copy(rbuf[p], out.at[c, pl.ds((s * _RCH + t) * _CHUNK, _CHUNK)])


_GRP = 22
_NGRP = _NCHUNKS // _GRP


def _agg_body(table, srcp, dstp, zrows, out,
              is0, is1, id0, id1, id2, rows0, rows1,
              sg0, sg1, si0, si1, ss0, ss1, acc):
    c = lax.axis_index("c")
    s = lax.axis_index("s")
    ebase = c * _EPC + s * _EPT
    pltpu.sync_copy(zrows, rows0)
    zd = [pltpu.async_copy(rows0, acc.at[pl.ds((s * _RCH + t) * _CHUNK, _CHUNK)],
                           sg0)
          for t in range(_RCH)]
    for d in zd:
        d.wait()
    plsc.subcore_barrier()

    isb = (is0, is1)
    idb = (id0, id1, id2)
    rb = (rows0, rows1)
    sg = (sg0, sg1)
    si = (si0, si1)
    ss = (ss0, ss1)

    def group(q, carry):
        gbase = ebase + q * _GRP * _CHUNK
        pltpu.sync_copy(srcp.at[pl.ds(gbase, _CHUNK)], is0)
        pltpu.sync_copy(dstp.at[pl.ds(gbase, _CHUNK)], id0)
        gd = [pltpu.async_copy(table.at[is0], rows0, sg0)]
        pref = {1: (pltpu.async_copy(srcp.at[pl.ds(gbase + _CHUNK, _CHUNK)], is1, si1),
                    pltpu.async_copy(dstp.at[pl.ds(gbase + _CHUNK, _CHUNK)], id1, si1))}
        sd = {}
        for k in range(_GRP):
            p = k & 1
            gd[k].wait()
            if k >= 1:
                sd[k - 1].wait()
            if k + 1 < _GRP:
                a, b = pref.pop(k + 1)
                a.wait()
                b.wait()
                gd.append(pltpu.async_copy(table.at[isb[1 - p]], rb[1 - p], sg[1 - p]))
            sd[k] = pltpu.async_copy(rb[p], acc.at[idb[k % 3]], ss[p], add=True)
            if k + 2 < _GRP:
                off = gbase + (k + 2) * _CHUNK
                pref[k + 2] = (
                    pltpu.async_copy(srcp.at[pl.ds(off, _CHUNK)], isb[p], si[p]),
                    pltpu.async_copy(dstp.at[pl.ds(off, _CHUNK)], idb[(k + 2) % 3], si[p]))
        sd[_GRP - 1].wait()
        return carry

    lax.fori_loop(0, _NGRP, group, 0)
    plsc.subcore_barrier()
    rd = pltpu.async_copy(acc.at[pl.ds(s * _RCH * _CHUNK, _CHUNK)], rows0, sg0)
    for t in range(_RCH):
        p = t & 1
        rd.wait()
        if t + 1 < _RCH:
            rd = pltpu.async_copy(
                acc.at[pl.ds((s * _RCH + t + 1) * _CHUNK, _CHUNK)],
                rb[1 - p], sg[1 - p])
        pltpu.sync_copy(rb[p], out.at[c, pl.ds((s * _RCH + t) * _CHUNK, _CHUNK)])


@functools.lru_cache(maxsize=None)
def _sc_kernels():
    mesh = plsc.VectorSubcoreMesh(core_axis_name="c", subcore_axis_name="s")
    deg = pl.kernel(
        _deg_body,
        out_type=jax.ShapeDtypeStruct((_NCORES, _NPAD, _D), jnp.float32),
        mesh=mesh,
        scratch_types=[
            pltpu.VMEM((_NCHUNKS, _CHUNK), jnp.int32),
            pltpu.VMEM((_CHUNK, _D), jnp.float32),
            pltpu.VMEM((_CHUNK, _D), jnp.float32),
            pltpu.SemaphoreType.DMA,
            pltpu.VMEM_SHARED((_NPAD, _D), jnp.float32),
        ],
    )
    agg = pl.kernel(
        _agg_body,
        out_type=jax.ShapeDtypeStruct((_NCORES, _NPAD, _D), jnp.float32),
        mesh=mesh,
        scratch_types=[
            pltpu.VMEM((_CHUNK,), jnp.int32),
            pltpu.VMEM((_CHUNK,), jnp.int32),
            pltpu.VMEM((_CHUNK,), jnp.int32),
            pltpu.VMEM((_CHUNK,), jnp.int32),
            pltpu.VMEM((_CHUNK,), jnp.int32),
            pltpu.VMEM((_CHUNK, _D), jnp.float32),
            pltpu.VMEM((_CHUNK, _D), jnp.float32),
            pltpu.SemaphoreType.DMA,
            pltpu.SemaphoreType.DMA,
            pltpu.SemaphoreType.DMA,
            pltpu.SemaphoreType.DMA,
            pltpu.SemaphoreType.DMA,
            pltpu.SemaphoreType.DMA,
            pltpu.VMEM_SHARED((_NPAD, _D), jnp.float32),
        ],
    )
    return deg, agg


def _k0_body(x_ref, w_ref, degp_ref, h_ref, dn_ref):
    deg = degp_ref[0, :_N, 0:1] + degp_ref[1, :_N, 0:1]
    dn = lax.rsqrt(jnp.maximum(deg, 1.0))
    h = jnp.dot(x_ref[...], w_ref[...], preferred_element_type=jnp.float32)
    h_ref[...] = h * dn
    dn_ref[...] = dn


_k0 = pl.pallas_call(
    _k0_body,
    out_shape=[
        jax.ShapeDtypeStruct((_N, _D), jnp.float32),
        jax.ShapeDtypeStruct((_N, 1), jnp.float32),
    ],
)


def _mid_body(aggp_ref, dn_ref, b_ref, g_ref, be_ref, w_ref, out_ref):
    a = aggp_ref[0, :_N, :] + aggp_ref[1, :_N, :]
    dn = dn_ref[...]
    r = jnp.maximum(a * dn + b_ref[...], 0.0)
    mu = jnp.mean(r, axis=0, keepdims=True)
    var = jnp.mean(r * r, axis=0, keepdims=True) - mu * mu
    y = (r - mu) * lax.rsqrt(var + _EPS) * g_ref[...] + be_ref[...]
    out_ref[...] = jnp.dot(y, w_ref[...], preferred_element_type=jnp.float32) * dn


_mid = pl.pallas_call(
    _mid_body,
    out_shape=jax.ShapeDtypeStruct((_N, _D), jnp.float32),
)


def _fin_body(aggp_ref, dn_ref, b_ref, g_ref, be_ref, batch_ref, clin_ref,
              wc1_ref, wc2_ref, bc_ref, out_ref):
    a = aggp_ref[0, :_N, :] + aggp_ref[1, :_N, :]
    r = jnp.maximum(a * dn_ref[...] + b_ref[...], 0.0)
    mu = jnp.mean(r, axis=0, keepdims=True)
    var = jnp.mean(r * r, axis=0, keepdims=True) - mu * mu
    y = (r - mu) * lax.rsqrt(var + _EPS) * g_ref[...] + be_ref[...]
    gids = lax.broadcasted_iota(jnp.int32, (_G, _N), 0)
    oh = jnp.where(gids == batch_ref[0:1, :_N], 1.0, 0.0)
    sums = jnp.dot(oh, y, preferred_element_type=jnp.float32)
    cnt = jnp.sum(oh, axis=1, keepdims=True)
    pooled = sums / jnp.maximum(cnt, 1.0)
    out_ref[...] = (jnp.dot(pooled, wc1_ref[...], preferred_element_type=jnp.float32)
                    + jnp.dot(clin_ref[...], wc2_ref[...], preferred_element_type=jnp.float32)
                    + bc_ref[...])


_fin = pl.pallas_call(
    _fin_body,
    out_shape=jax.ShapeDtypeStruct((_G, _D), jnp.float32),
)


def kernel(x, edge_index, batch, clinical, W1, b1, g1, be1, W2, b2, g2, be2,
           W3, b3, g3, be3, Wc, bc):
    loop = jnp.arange(_N, dtype=jnp.int32)
    npad_e = _EPAD - (_E + _N)
    padi = jnp.arange(npad_e, dtype=jnp.int32)
    srcp = jnp.concatenate([edge_index[0], loop, padi % _N])
    dstp = jnp.concatenate([edge_index[1], loop, _N + padi % (_NPAD - _N)])
    dstp2 = dstp.reshape(-1, _CHUNK)
    batchp = jnp.concatenate(
        [batch, jnp.full((_NPAD - _N,), _G, jnp.int32)]).reshape(1, _NPAD)
    zrows = jnp.zeros((_CHUNK, _D), jnp.float32)
    onesr = jnp.ones((_CHUNK, _D), jnp.float32)
    wc1 = jnp.pad(Wc[:_D], ((0, 0), (0, _D - Wc.shape[1])))
    wc2 = jnp.pad(Wc[_D:], ((0, 0), (0, _D - Wc.shape[1])))
    bcp = jnp.pad(bc, (0, _D - bc.shape[0]))

    _deg_kernel, _agg_kernel = _sc_kernels()
    degp = _deg_kernel(dstp2, onesr, zrows)
    h1p, dn = _k0(x, W1, degp)
    a1 = _agg_kernel(h1p, srcp, dstp, zrows)
    h2p = _mid(a1, dn, b1, g1, be1, W2)
    a2 = _agg_kernel(h2p, srcp, dstp, zrows)
    h3p = _mid(a2, dn, b2, g2, be2, W3)
    a3 = _agg_kernel(h3p, srcp, dstp, zrows)
    o = _fin(a3, dn, b3, g3, be3, batchp, clinical, wc1, wc2, bcp)
    return o[:, :Wc.shape[1]]


# speedup vs baseline: 20.9752x; 1.0228x over previous
import functools

import jax
import jax.numpy as jnp
from jax import lax
from jax.experimental import pallas as pl
from jax.experimental.pallas import tpu as pltpu
from jax.experimental.pallas import tpu_sc as plsc

_N = 10000
_NPAD = 10240
_D = 128
_E = 320000
_EPAD = 360448
_CHUNK = 128
_NCORES = 2
_NSUB = 16
_EPC = _EPAD // _NCORES
_EPT = _EPC // _NSUB
_NCHUNKS = _EPT // _CHUNK
_RPT = _NPAD // _NSUB
_RCH = _RPT // _CHUNK
_EPS = 1e-5
_G = 64
_NC = 16

def _deg_body(dstp, onesr, zrows, out, idx_d, ones_v, zb, sem, acc):
    c = lax.axis_index("c")
    s = lax.axis_index("s")
    cb = (c * _NSUB + s) * _NCHUNKS
    pltpu.sync_copy(dstp.at[pl.ds(cb, _NCHUNKS)], idx_d)
    pltpu.sync_copy(onesr, ones_v)
    pltpu.sync_copy(zrows, zb)
    zd = [pltpu.async_copy(zb, acc.at[pl.ds((s * _RCH + t) * _CHUNK, _CHUNK)],
                           sem)
          for t in range(_RCH)]
    for d in zd:
        d.wait()
    plsc.subcore_barrier()

    def step(q, carry):
        ds = [pltpu.async_copy(ones_v, acc.at[idx_d.at[q * 8 + j]], sem,
                               add=True)
              for j in range(8)]
        for d in ds:
            d.wait()
        return carry

    lax.fori_loop(0, _NCHUNKS // 8, step, 0)
    plsc.subcore_barrier()
    rbuf = (zb, ones_v)
    rd = pltpu.async_copy(acc.at[pl.ds(s * _RCH * _CHUNK, _CHUNK)], zb, sem)
    for t in range(_RCH):
        p = t & 1
        rd.wait()
        if t + 1 < _RCH:
            rd = pltpu.async_copy(
                acc.at[pl.ds((s * _RCH + t + 1) * _CHUNK, _CHUNK)],
                rbuf[1 - p], sem)
        pltpu.sync_copy(rbuf[p], out.at[c, pl.ds((s * _RCH + t) * _CHUNK, _CHUNK)])


_GRP = 88
_NGRP = _NCHUNKS // _GRP


def _agg_body(table, srcp, dstp, zrows, out,
              is0, is1, id0, id1, id2, rows0, rows1,
              sg0, sg1, si0, si1, ss0, ss1, acc):
    c = lax.axis_index("c")
    s = lax.axis_index("s")
    ebase = c * _EPC + s * _EPT
    pltpu.sync_copy(zrows, rows0)
    zd = [pltpu.async_copy(rows0, acc.at[pl.ds((s * _RCH + t) * _CHUNK, _CHUNK)],
                           sg0)
          for t in range(_RCH)]
    for d in zd:
        d.wait()
    plsc.subcore_barrier()

    isb = (is0, is1)
    idb = (id0, id1, id2)
    rb = (rows0, rows1)
    sg = (sg0, sg1)
    si = (si0, si1)
    ss = (ss0, ss1)

    def group(q, carry):
        gbase = ebase + q * _GRP * _CHUNK
        pltpu.sync_copy(srcp.at[pl.ds(gbase, _CHUNK)], is0)
        pltpu.sync_copy(dstp.at[pl.ds(gbase, _CHUNK)], id0)
        gd = [pltpu.async_copy(table.at[is0], rows0, sg0)]
        pref = {1: (pltpu.async_copy(srcp.at[pl.ds(gbase + _CHUNK, _CHUNK)], is1, si1),
                    pltpu.async_copy(dstp.at[pl.ds(gbase + _CHUNK, _CHUNK)], id1, si1))}
        sd = {}
        for k in range(_GRP):
            p = k & 1
            gd[k].wait()
            if k >= 1:
                sd[k - 1].wait()
            if k + 1 < _GRP:
                a, b = pref.pop(k + 1)
                a.wait()
                b.wait()
                gd.append(pltpu.async_copy(table.at[isb[1 - p]], rb[1 - p], sg[1 - p]))
            sd[k] = pltpu.async_copy(rb[p], acc.at[idb[k % 3]], ss[p], add=True)
            if k + 2 < _GRP:
                off = gbase + (k + 2) * _CHUNK
                pref[k + 2] = (
                    pltpu.async_copy(srcp.at[pl.ds(off, _CHUNK)], isb[p], si[p]),
                    pltpu.async_copy(dstp.at[pl.ds(off, _CHUNK)], idb[(k + 2) % 3], si[p]))
        sd[_GRP - 1].wait()
        return carry

    lax.fori_loop(0, _NGRP, group, 0)
    plsc.subcore_barrier()
    rd = pltpu.async_copy(acc.at[pl.ds(s * _RCH * _CHUNK, _CHUNK)], rows0, sg0)
    for t in range(_RCH):
        p = t & 1
        rd.wait()
        if t + 1 < _RCH:
            rd = pltpu.async_copy(
                acc.at[pl.ds((s * _RCH + t + 1) * _CHUNK, _CHUNK)],
                rb[1 - p], sg[1 - p])
        pltpu.sync_copy(rb[p], out.at[c, pl.ds((s * _RCH + t) * _CHUNK, _CHUNK)])


@functools.lru_cache(maxsize=None)
def _sc_kernels():
    mesh = plsc.VectorSubcoreMesh(core_axis_name="c", subcore_axis_name="s")
    deg = pl.kernel(
        _deg_body,
        out_type=jax.ShapeDtypeStruct((_NCORES, _NPAD, _D), jnp.float32),
        mesh=mesh,
        scratch_types=[
            pltpu.VMEM((_NCHUNKS, _CHUNK), jnp.int32),
            pltpu.VMEM((_CHUNK, _D), jnp.float32),
            pltpu.VMEM((_CHUNK, _D), jnp.float32),
            pltpu.SemaphoreType.DMA,
            pltpu.VMEM_SHARED((_NPAD, _D), jnp.float32),
        ],
    )
    agg = pl.kernel(
        _agg_body,
        out_type=jax.ShapeDtypeStruct((_NCORES, _NPAD, _D), jnp.float32),
        mesh=mesh,
        scratch_types=[
            pltpu.VMEM((_CHUNK,), jnp.int32),
            pltpu.VMEM((_CHUNK,), jnp.int32),
            pltpu.VMEM((_CHUNK,), jnp.int32),
            pltpu.VMEM((_CHUNK,), jnp.int32),
            pltpu.VMEM((_CHUNK,), jnp.int32),
            pltpu.VMEM((_CHUNK, _D), jnp.float32),
            pltpu.VMEM((_CHUNK, _D), jnp.float32),
            pltpu.SemaphoreType.DMA,
            pltpu.SemaphoreType.DMA,
            pltpu.SemaphoreType.DMA,
            pltpu.SemaphoreType.DMA,
            pltpu.SemaphoreType.DMA,
            pltpu.SemaphoreType.DMA,
            pltpu.VMEM_SHARED((_NPAD, _D), jnp.float32),
        ],
    )
    return deg, agg


def _k0_body(x_ref, w_ref, degp_ref, h_ref, dn_ref):
    deg = degp_ref[0, :_N, 0:1] + degp_ref[1, :_N, 0:1]
    dn = lax.rsqrt(jnp.maximum(deg, 1.0))
    h = jnp.dot(x_ref[...], w_ref[...], preferred_element_type=jnp.float32)
    h_ref[...] = h * dn
    dn_ref[...] = dn


_k0 = pl.pallas_call(
    _k0_body,
    out_shape=[
        jax.ShapeDtypeStruct((_N, _D), jnp.float32),
        jax.ShapeDtypeStruct((_N, 1), jnp.float32),
    ],
)


def _mid_body(aggp_ref, dn_ref, b_ref, g_ref, be_ref, w_ref, out_ref):
    a = aggp_ref[0, :_N, :] + aggp_ref[1, :_N, :]
    dn = dn_ref[...]
    r = jnp.maximum(a * dn + b_ref[...], 0.0)
    mu = jnp.mean(r, axis=0, keepdims=True)
    var = jnp.mean(r * r, axis=0, keepdims=True) - mu * mu
    y = (r - mu) * lax.rsqrt(var + _EPS) * g_ref[...] + be_ref[...]
    out_ref[...] = jnp.dot(y, w_ref[...], preferred_element_type=jnp.float32) * dn


_mid = pl.pallas_call(
    _mid_body,
    out_shape=jax.ShapeDtypeStruct((_N, _D), jnp.float32),
)


def _fin_body(aggp_ref, dn_ref, b_ref, g_ref, be_ref, batch_ref, clin_ref,
              wc1_ref, wc2_ref, bc_ref, out_ref):
    a = aggp_ref[0, :_N, :] + aggp_ref[1, :_N, :]
    r = jnp.maximum(a * dn_ref[...] + b_ref[...], 0.0)
    mu = jnp.mean(r, axis=0, keepdims=True)
    var = jnp.mean(r * r, axis=0, keepdims=True) - mu * mu
    y = (r - mu) * lax.rsqrt(var + _EPS) * g_ref[...] + be_ref[...]
    gids = lax.broadcasted_iota(jnp.int32, (_G, _N), 0)
    oh = jnp.where(gids == batch_ref[0:1, :_N], 1.0, 0.0)
    sums = jnp.dot(oh, y, preferred_element_type=jnp.float32)
    cnt = jnp.sum(oh, axis=1, keepdims=True)
    pooled = sums / jnp.maximum(cnt, 1.0)
    out_ref[...] = (jnp.dot(pooled, wc1_ref[...], preferred_element_type=jnp.float32)
                    + jnp.dot(clin_ref[...], wc2_ref[...], preferred_element_type=jnp.float32)
                    + bc_ref[...])


_fin = pl.pallas_call(
    _fin_body,
    out_shape=jax.ShapeDtypeStruct((_G, _D), jnp.float32),
)


def kernel(x, edge_index, batch, clinical, W1, b1, g1, be1, W2, b2, g2, be2,
           W3, b3, g3, be3, Wc, bc):
    loop = jnp.arange(_N, dtype=jnp.int32)
    npad_e = _EPAD - (_E + _N)
    padi = jnp.arange(npad_e, dtype=jnp.int32)
    srcp = jnp.concatenate([edge_index[0], loop, padi % _N])
    dstp = jnp.concatenate([edge_index[1], loop, _N + padi % (_NPAD - _N)])
    dstp2 = dstp.reshape(-1, _CHUNK)
    batchp = jnp.concatenate(
        [batch, jnp.full((_NPAD - _N,), _G, jnp.int32)]).reshape(1, _NPAD)
    zrows = jnp.zeros((_CHUNK, _D), jnp.float32)
    onesr = jnp.ones((_CHUNK, _D), jnp.float32)
    wc1 = jnp.pad(Wc[:_D], ((0, 0), (0, _D - Wc.shape[1])))
    wc2 = jnp.pad(Wc[_D:], ((0, 0), (0, _D - Wc.shape[1])))
    bcp = jnp.pad(bc, (0, _D - bc.shape[0]))

    _deg_kernel, _agg_kernel = _sc_kernels()
    degp = _deg_kernel(dstp2, onesr, zrows)
    h1p, dn = _k0(x, W1, degp)
    a1 = _agg_kernel(h1p, srcp, dstp, zrows)
    h2p = _mid(a1, dn, b1, g1, be1, W2)
    a2 = _agg_kernel(h2p, srcp, dstp, zrows)
    h3p = _mid(a2, dn, b2, g2, be2, W3)
    a3 = _agg_kernel(h3p, srcp, dstp, zrows)
    o = _fin(a3, dn, b3, g3, be3, batchp, clinical, wc1, wc2, bcp)
    return o[:, :Wc.shape[1]]


# speedup vs baseline: 23.8710x; 1.1381x over previous
import functools

import jax
import jax.numpy as jnp
from jax import lax
from jax.experimental import pallas as pl
from jax.experimental.pallas import tpu as pltpu
from jax.experimental.pallas import tpu_sc as plsc

_N = 10000
_NPAD = 10240
_D = 128
_E = 320000
_EPAD = 360448
_CHUNK = 128
_NCORES = 2
_NSUB = 16
_EPC = _EPAD // _NCORES
_EPT = _EPC // _NSUB
_NCHUNKS = _EPT // _CHUNK
_RPT = _NPAD // _NSUB
_RCH = _RPT // _CHUNK
_EPS = 1e-5
_G = 64
_NC = 16

def _deg_body(dstp, ones1, zvec, out, idx_d, ones_v, stage, sem, acc):
    c = lax.axis_index("c")
    s = lax.axis_index("s")
    cb = (c * _NSUB + s) * _NCHUNKS
    pltpu.sync_copy(dstp.at[pl.ds(cb, _NCHUNKS)], idx_d)
    pltpu.sync_copy(ones1, ones_v)

    @pl.when(s == 0)
    def _():
        pltpu.sync_copy(zvec, acc)

    plsc.subcore_barrier()

    def step(q, carry):
        ds = [pltpu.async_copy(ones_v, acc.at[idx_d.at[q * 8 + j]], sem,
                               add=True)
              for j in range(8)]
        for d in ds:
            d.wait()
        return carry

    lax.fori_loop(0, _NCHUNKS // 8, step, 0)
    plsc.subcore_barrier()

    @pl.when(s == 0)
    def _():
        pltpu.sync_copy(acc, stage)
        pltpu.sync_copy(stage, out.at[c])


_GRP = 88
_NGRP = _NCHUNKS // _GRP


def _agg_body(table, srcp, dstp, zrows, out,
              is0, is1, id0, id1, id2, rows0, rows1,
              sg0, sg1, si0, si1, ss0, ss1, acc):
    c = lax.axis_index("c")
    s = lax.axis_index("s")
    ebase = c * _EPC + s * _EPT
    pltpu.sync_copy(zrows, rows0)
    zd = [pltpu.async_copy(rows0, acc.at[pl.ds((s * _RCH + t) * _CHUNK, _CHUNK)],
                           sg0)
          for t in range(_RCH)]
    for d in zd:
        d.wait()
    plsc.subcore_barrier()

    isb = (is0, is1)
    idb = (id0, id1, id2)
    rb = (rows0, rows1)
    sg = (sg0, sg1)
    si = (si0, si1)
    ss = (ss0, ss1)

    def group(q, carry):
        gbase = ebase + q * _GRP * _CHUNK
        pltpu.sync_copy(srcp.at[pl.ds(gbase, _CHUNK)], is0)
        pltpu.sync_copy(dstp.at[pl.ds(gbase, _CHUNK)], id0)
        gd = [pltpu.async_copy(table.at[is0], rows0, sg0)]
        pref = {1: (pltpu.async_copy(srcp.at[pl.ds(gbase + _CHUNK, _CHUNK)], is1, si1),
                    pltpu.async_copy(dstp.at[pl.ds(gbase + _CHUNK, _CHUNK)], id1, si1))}
        sd = {}
        for k in range(_GRP):
            p = k & 1
            gd[k].wait()
            if k >= 1:
                sd[k - 1].wait()
            if k + 1 < _GRP:
                a, b = pref.pop(k + 1)
                a.wait()
                b.wait()
                gd.append(pltpu.async_copy(table.at[isb[1 - p]], rb[1 - p], sg[1 - p]))
            sd[k] = pltpu.async_copy(rb[p], acc.at[idb[k % 3]], ss[p], add=True)
            if k + 2 < _GRP:
                off = gbase + (k + 2) * _CHUNK
                pref[k + 2] = (
                    pltpu.async_copy(srcp.at[pl.ds(off, _CHUNK)], isb[p], si[p]),
                    pltpu.async_copy(dstp.at[pl.ds(off, _CHUNK)], idb[(k + 2) % 3], si[p]))
        sd[_GRP - 1].wait()
        return carry

    lax.fori_loop(0, _NGRP, group, 0)
    plsc.subcore_barrier()
    rd = pltpu.async_copy(acc.at[pl.ds(s * _RCH * _CHUNK, _CHUNK)], rows0, sg0)
    for t in range(_RCH):
        p = t & 1
        rd.wait()
        if t + 1 < _RCH:
            rd = pltpu.async_copy(
                acc.at[pl.ds((s * _RCH + t + 1) * _CHUNK, _CHUNK)],
                rb[1 - p], sg[1 - p])
        pltpu.sync_copy(rb[p], out.at[c, pl.ds((s * _RCH + t) * _CHUNK, _CHUNK)])


@functools.lru_cache(maxsize=None)
def _sc_kernels():
    mesh = plsc.VectorSubcoreMesh(core_axis_name="c", subcore_axis_name="s")
    deg = pl.kernel(
        _deg_body,
        out_type=jax.ShapeDtypeStruct((_NCORES, _NPAD), jnp.float32),
        mesh=mesh,
        scratch_types=[
            pltpu.VMEM((_NCHUNKS, _CHUNK), jnp.int32),
            pltpu.VMEM((_CHUNK,), jnp.float32),
            pltpu.VMEM((_NPAD,), jnp.float32),
            pltpu.SemaphoreType.DMA,
            pltpu.VMEM_SHARED((_NPAD,), jnp.float32),
        ],
    )
    agg = pl.kernel(
        _agg_body,
        out_type=jax.ShapeDtypeStruct((_NCORES, _NPAD, _D), jnp.float32),
        mesh=mesh,
        scratch_types=[
            pltpu.VMEM((_CHUNK,), jnp.int32),
            pltpu.VMEM((_CHUNK,), jnp.int32),
            pltpu.VMEM((_CHUNK,), jnp.int32),
            pltpu.VMEM((_CHUNK,), jnp.int32),
            pltpu.VMEM((_CHUNK,), jnp.int32),
            pltpu.VMEM((_CHUNK, _D), jnp.float32),
            pltpu.VMEM((_CHUNK, _D), jnp.float32),
            pltpu.SemaphoreType.DMA,
            pltpu.SemaphoreType.DMA,
            pltpu.SemaphoreType.DMA,
            pltpu.SemaphoreType.DMA,
            pltpu.SemaphoreType.DMA,
            pltpu.SemaphoreType.DMA,
            pltpu.VMEM_SHARED((_NPAD, _D), jnp.float32),
        ],
    )
    return deg, agg


def _k0_body(x_ref, w_ref, degp_ref, h_ref, dn_ref):
    deg = lax.dot_general(degp_ref[...], jnp.ones((_NCORES, 1), jnp.float32),
                          (((0,), (0,)), ((), ())),
                          preferred_element_type=jnp.float32)[:_N]
    dn = lax.rsqrt(jnp.maximum(deg, 1.0))
    h = jnp.dot(x_ref[...], w_ref[...], preferred_element_type=jnp.float32)
    h_ref[...] = h * dn
    dn_ref[...] = dn


_k0 = pl.pallas_call(
    _k0_body,
    out_shape=[
        jax.ShapeDtypeStruct((_N, _D), jnp.float32),
        jax.ShapeDtypeStruct((_N, 1), jnp.float32),
    ],
)


def _mid_body(aggp_ref, dn_ref, b_ref, g_ref, be_ref, w_ref, out_ref):
    a = aggp_ref[0, :_N, :] + aggp_ref[1, :_N, :]
    dn = dn_ref[...]
    r = jnp.maximum(a * dn + b_ref[...], 0.0)
    mu = jnp.mean(r, axis=0, keepdims=True)
    var = jnp.mean(r * r, axis=0, keepdims=True) - mu * mu
    y = (r - mu) * lax.rsqrt(var + _EPS) * g_ref[...] + be_ref[...]
    out_ref[...] = jnp.dot(y, w_ref[...], preferred_element_type=jnp.float32) * dn


_mid = pl.pallas_call(
    _mid_body,
    out_shape=jax.ShapeDtypeStruct((_N, _D), jnp.float32),
)


def _fin_body(aggp_ref, dn_ref, b_ref, g_ref, be_ref, batch_ref, clin_ref,
              wc1_ref, wc2_ref, bc_ref, out_ref):
    a = aggp_ref[0, :_N, :] + aggp_ref[1, :_N, :]
    r = jnp.maximum(a * dn_ref[...] + b_ref[...], 0.0)
    mu = jnp.mean(r, axis=0, keepdims=True)
    var = jnp.mean(r * r, axis=0, keepdims=True) - mu * mu
    y = (r - mu) * lax.rsqrt(var + _EPS) * g_ref[...] + be_ref[...]
    gids = lax.broadcasted_iota(jnp.int32, (_G, _N), 0)
    oh = jnp.where(gids == batch_ref[0:1, :_N], 1.0, 0.0)
    sums = jnp.dot(oh, y, preferred_element_type=jnp.float32)
    cnt = jnp.sum(oh, axis=1, keepdims=True)
    pooled = sums / jnp.maximum(cnt, 1.0)
    out_ref[...] = (jnp.dot(pooled, wc1_ref[...], preferred_element_type=jnp.float32)
                    + jnp.dot(clin_ref[...], wc2_ref[...], preferred_element_type=jnp.float32)
                    + bc_ref[...])


_fin = pl.pallas_call(
    _fin_body,
    out_shape=jax.ShapeDtypeStruct((_G, _D), jnp.float32),
)


def kernel(x, edge_index, batch, clinical, W1, b1, g1, be1, W2, b2, g2, be2,
           W3, b3, g3, be3, Wc, bc):
    loop = jnp.arange(_N, dtype=jnp.int32)
    npad_e = _EPAD - (_E + _N)
    padi = jnp.arange(npad_e, dtype=jnp.int32)
    srcp = jnp.concatenate([edge_index[0], loop, padi % _N])
    dstp = jnp.concatenate([edge_index[1], loop, _N + padi % (_NPAD - _N)])
    dstp2 = dstp.reshape(-1, _CHUNK)
    batchp = jnp.concatenate(
        [batch, jnp.full((_NPAD - _N,), _G, jnp.int32)]).reshape(1, _NPAD)
    zrows = jnp.zeros((_CHUNK, _D), jnp.float32)
    ones1 = jnp.ones((_CHUNK,), jnp.float32)
    zvec = jnp.zeros((_NPAD,), jnp.float32)
    wc1 = jnp.pad(Wc[:_D], ((0, 0), (0, _D - Wc.shape[1])))
    wc2 = jnp.pad(Wc[_D:], ((0, 0), (0, _D - Wc.shape[1])))
    bcp = jnp.pad(bc, (0, _D - bc.shape[0]))

    _deg_kernel, _agg_kernel = _sc_kernels()
    degp = _deg_kernel(dstp2, ones1, zvec)
    h1p, dn = _k0(x, W1, degp)
    a1 = _agg_kernel(h1p, srcp, dstp, zrows)
    h2p = _mid(a1, dn, b1, g1, be1, W2)
    a2 = _agg_kernel(h2p, srcp, dstp, zrows)
    h3p = _mid(a2, dn, b2, g2, be2, W3)
    a3 = _agg_kernel(h3p, srcp, dstp, zrows)
    o = _fin(a3, dn, b3, g3, be3, batchp, clinical, wc1, wc2, bcp)
    return o[:, :Wc.shape[1]]


# speedup vs baseline: 25.4974x; 1.0681x over previous
import functools

import jax
import jax.numpy as jnp
from jax import lax
from jax.experimental import pallas as pl
from jax.experimental.pallas import tpu as pltpu
from jax.experimental.pallas import tpu_sc as plsc

_N = 10000
_NPAD = 10240
_D = 128
_E = 320000
_EPAD = 327680
_CHUNK = 128
_NCORES = 2
_NSUB = 16
_EPC = _EPAD // _NCORES
_EPT = _EPC // _NSUB
_NCHUNKS = _EPT // _CHUNK
_RPT = _NPAD // _NSUB
_RCH = _RPT // _CHUNK
_EPS = 1e-5
_G = 64
_NC = 16

def _deg_body(dstp, ones1, zvec, out, idx_d, ones_v, stage, sem, acc):
    c = lax.axis_index("c")
    s = lax.axis_index("s")
    cb = (c * _NSUB + s) * _NCHUNKS
    pltpu.sync_copy(dstp.at[pl.ds(cb, _NCHUNKS)], idx_d)
    pltpu.sync_copy(ones1, ones_v)

    @pl.when(s == 0)
    def _():
        pltpu.sync_copy(zvec, acc)

    plsc.subcore_barrier()

    def step(q, carry):
        ds = [pltpu.async_copy(ones_v, acc.at[idx_d.at[q * 8 + j]], sem,
                               add=True)
              for j in range(8)]
        for d in ds:
            d.wait()
        return carry

    lax.fori_loop(0, _NCHUNKS // 8, step, 0)
    plsc.subcore_barrier()

    @pl.when(s == 0)
    def _():
        pltpu.sync_copy(acc, stage)
        pltpu.sync_copy(stage, out.at[c])


_GRP = 80
_NGRP = _NCHUNKS // _GRP


def _agg_body(table, srcp, dstp, zrows, out,
              is0, is1, id0, id1, id2, rows0, rows1,
              sg0, sg1, si0, si1, ss0, ss1, acc):
    c = lax.axis_index("c")
    s = lax.axis_index("s")
    ebase = c * _EPC + s * _EPT
    pltpu.sync_copy(zrows, rows0)
    zd = [pltpu.async_copy(rows0, acc.at[pl.ds((s * _RCH + t) * _CHUNK, _CHUNK)],
                           sg0)
          for t in range(_RCH)]
    for d in zd:
        d.wait()
    plsc.subcore_barrier()

    isb = (is0, is1)
    idb = (id0, id1, id2)
    rb = (rows0, rows1)
    sg = (sg0, sg1)
    si = (si0, si1)
    ss = (ss0, ss1)

    def group(q, carry):
        gbase = ebase + q * _GRP * _CHUNK
        pltpu.sync_copy(srcp.at[pl.ds(gbase, _CHUNK)], is0)
        pltpu.sync_copy(dstp.at[pl.ds(gbase, _CHUNK)], id0)
        gd = [pltpu.async_copy(table.at[is0], rows0, sg0)]
        pref = {1: (pltpu.async_copy(srcp.at[pl.ds(gbase + _CHUNK, _CHUNK)], is1, si1),
                    pltpu.async_copy(dstp.at[pl.ds(gbase + _CHUNK, _CHUNK)], id1, si1))}
        sd = {}
        for k in range(_GRP):
            p = k & 1
            gd[k].wait()
            if k >= 1:
                sd[k - 1].wait()
            if k + 1 < _GRP:
                a, b = pref.pop(k + 1)
                a.wait()
                b.wait()
                gd.append(pltpu.async_copy(table.at[isb[1 - p]], rb[1 - p], sg[1 - p]))
            sd[k] = pltpu.async_copy(rb[p], acc.at[idb[k % 3]], ss[p], add=True)
            if k + 2 < _GRP:
                off = gbase + (k + 2) * _CHUNK
                pref[k + 2] = (
                    pltpu.async_copy(srcp.at[pl.ds(off, _CHUNK)], isb[p], si[p]),
                    pltpu.async_copy(dstp.at[pl.ds(off, _CHUNK)], idb[(k + 2) % 3], si[p]))
        sd[_GRP - 1].wait()
        return carry

    lax.fori_loop(0, _NGRP, group, 0)
    plsc.subcore_barrier()
    rd = pltpu.async_copy(acc.at[pl.ds(s * _RCH * _CHUNK, _CHUNK)], rows0, sg0)
    for t in range(_RCH):
        p = t & 1
        rd.wait()
        if t + 1 < _RCH:
            rd = pltpu.async_copy(
                acc.at[pl.ds((s * _RCH + t + 1) * _CHUNK, _CHUNK)],
                rb[1 - p], sg[1 - p])
        pltpu.sync_copy(rb[p], out.at[c, pl.ds((s * _RCH + t) * _CHUNK, _CHUNK)])


@functools.lru_cache(maxsize=None)
def _sc_kernels():
    mesh = plsc.VectorSubcoreMesh(core_axis_name="c", subcore_axis_name="s")
    deg = pl.kernel(
        _deg_body,
        out_type=jax.ShapeDtypeStruct((_NCORES, _NPAD), jnp.float32),
        mesh=mesh,
        scratch_types=[
            pltpu.VMEM((_NCHUNKS, _CHUNK), jnp.int32),
            pltpu.VMEM((_CHUNK,), jnp.float32),
            pltpu.VMEM((_NPAD,), jnp.float32),
            pltpu.SemaphoreType.DMA,
            pltpu.VMEM_SHARED((_NPAD,), jnp.float32),
        ],
    )
    agg = pl.kernel(
        _agg_body,
        out_type=jax.ShapeDtypeStruct((_NCORES, _NPAD, _D), jnp.float32),
        mesh=mesh,
        scratch_types=[
            pltpu.VMEM((_CHUNK,), jnp.int32),
            pltpu.VMEM((_CHUNK,), jnp.int32),
            pltpu.VMEM((_CHUNK,), jnp.int32),
            pltpu.VMEM((_CHUNK,), jnp.int32),
            pltpu.VMEM((_CHUNK,), jnp.int32),
            pltpu.VMEM((_CHUNK, _D), jnp.float32),
            pltpu.VMEM((_CHUNK, _D), jnp.float32),
            pltpu.SemaphoreType.DMA,
            pltpu.SemaphoreType.DMA,
            pltpu.SemaphoreType.DMA,
            pltpu.SemaphoreType.DMA,
            pltpu.SemaphoreType.DMA,
            pltpu.SemaphoreType.DMA,
            pltpu.VMEM_SHARED((_NPAD, _D), jnp.float32),
        ],
    )
    return deg, agg


def _k0_body(x_ref, w_ref, degp_ref, h_ref, dn_ref):
    deg = lax.dot_general(degp_ref[...], jnp.ones((_NCORES, 1), jnp.float32),
                          (((0,), (0,)), ((), ())),
                          preferred_element_type=jnp.float32)[:_N] + 1.0
    dn = lax.rsqrt(jnp.maximum(deg, 1.0))
    h = jnp.dot(x_ref[...], w_ref[...], preferred_element_type=jnp.float32)
    h_ref[...] = h * dn
    dn_ref[...] = dn


_k0 = pl.pallas_call(
    _k0_body,
    out_shape=[
        jax.ShapeDtypeStruct((_N, _D), jnp.float32),
        jax.ShapeDtypeStruct((_N, 1), jnp.float32),
    ],
)


def _mid_body(aggp_ref, hprev_ref, dn_ref, b_ref, g_ref, be_ref, w_ref, out_ref):
    a = aggp_ref[0, :_N, :] + aggp_ref[1, :_N, :] + hprev_ref[...]
    dn = dn_ref[...]
    r = jnp.maximum(a * dn + b_ref[...], 0.0)
    mu = jnp.mean(r, axis=0, keepdims=True)
    var = jnp.mean(r * r, axis=0, keepdims=True) - mu * mu
    y = (r - mu) * lax.rsqrt(var + _EPS) * g_ref[...] + be_ref[...]
    out_ref[...] = jnp.dot(y, w_ref[...], preferred_element_type=jnp.float32) * dn


_mid = pl.pallas_call(
    _mid_body,
    out_shape=jax.ShapeDtypeStruct((_N, _D), jnp.float32),
)


def _fin_body(aggp_ref, hprev_ref, dn_ref, b_ref, g_ref, be_ref, batch_ref,
              clin_ref, wc1_ref, wc2_ref, bc_ref, out_ref):
    a = aggp_ref[0, :_N, :] + aggp_ref[1, :_N, :] + hprev_ref[...]
    r = jnp.maximum(a * dn_ref[...] + b_ref[...], 0.0)
    mu = jnp.mean(r, axis=0, keepdims=True)
    var = jnp.mean(r * r, axis=0, keepdims=True) - mu * mu
    y = (r - mu) * lax.rsqrt(var + _EPS) * g_ref[...] + be_ref[...]
    gids = lax.broadcasted_iota(jnp.int32, (_G, _N), 0)
    oh = jnp.where(gids == batch_ref[0:1, :_N], 1.0, 0.0)
    sums = jnp.dot(oh, y, preferred_element_type=jnp.float32)
    cnt = jnp.sum(oh, axis=1, keepdims=True)
    pooled = sums / jnp.maximum(cnt, 1.0)
    out_ref[...] = (jnp.dot(pooled, wc1_ref[...], preferred_element_type=jnp.float32)
                    + jnp.dot(clin_ref[...], wc2_ref[...], preferred_element_type=jnp.float32)
                    + bc_ref[...])


_fin = pl.pallas_call(
    _fin_body,
    out_shape=jax.ShapeDtypeStruct((_G, _D), jnp.float32),
)


def kernel(x, edge_index, batch, clinical, W1, b1, g1, be1, W2, b2, g2, be2,
           W3, b3, g3, be3, Wc, bc):
    npad_e = _EPAD - _E
    padi = jnp.arange(npad_e, dtype=jnp.int32)
    srcp = jnp.concatenate([edge_index[0], padi % _N])
    dstp = jnp.concatenate([edge_index[1], _N + padi % (_NPAD - _N)])
    dstp2 = dstp.reshape(-1, _CHUNK)
    batchp = jnp.concatenate(
        [batch, jnp.full((_NPAD - _N,), _G, jnp.int32)]).reshape(1, _NPAD)
    zrows = jnp.zeros((_CHUNK, _D), jnp.float32)
    ones1 = jnp.ones((_CHUNK,), jnp.float32)
    zvec = jnp.zeros((_NPAD,), jnp.float32)
    wc1 = jnp.pad(Wc[:_D], ((0, 0), (0, _D - Wc.shape[1])))
    wc2 = jnp.pad(Wc[_D:], ((0, 0), (0, _D - Wc.shape[1])))
    bcp = jnp.pad(bc, (0, _D - bc.shape[0]))

    _deg_kernel, _agg_kernel = _sc_kernels()
    degp = _deg_kernel(dstp2, ones1, zvec)
    h1p, dn = _k0(x, W1, degp)
    a1 = _agg_kernel(h1p, srcp, dstp, zrows)
    h2p = _mid(a1, h1p, dn, b1, g1, be1, W2)
    a2 = _agg_kernel(h2p, srcp, dstp, zrows)
    h3p = _mid(a2, h2p, dn, b2, g2, be2, W3)
    a3 = _agg_kernel(h3p, srcp, dstp, zrows)
    o = _fin(a3, h3p, dn, b3, g3, be3, batchp, clinical, wc1, wc2, bcp)
    return o[:, :Wc.shape[1]]
